# baseline (speedup 1.0000x reference)
import functools

import jax
import jax.numpy as jnp
from jax import lax
from jax.experimental import pallas as pl
from jax.experimental.pallas import tpu as pltpu
from jax.experimental.pallas import tpu_sc as plsc

N_M = 5000
N_G = 5000
N = N_M + N_G
E = 320000
R = 8
NB = 10
INIT = 256
IN = 128
HID = 128
OUT = 64

NC = 2
NS = 16
LANES = 16
NW = NC * NS
CHUNK = 80
EPT32 = E // NW
NCH32 = EPT32 // CHUNK
EPT16 = E // NS
NCH16 = EPT16 // CHUNK
KROWS = 5120
TBASE = N * R
NPAD = 10240
RPS = NPAD // NS
ZB = 128
_BR = 1000

_MESH = plsc.VectorSubcoreMesh(core_axis_name="c", subcore_axis_name="s")
_SC_PARAMS = pltpu.CompilerParams(needs_layout_passes=False,
                                  use_tc_tiling_on_sc=False)


@functools.partial(
    pl.kernel,
    out_type=jax.ShapeDtypeStruct((NW, KROWS * 16), jnp.float32),
    mesh=_MESH,
    compiler_params=_SC_PARAMS,
    scratch_types=[
        pltpu.VMEM((NCH32, CHUNK), jnp.int32),
        pltpu.VMEM((NCH32, CHUNK), jnp.int32),
        pltpu.VMEM((KROWS * 16,), jnp.float32),
    ],
)
def _sc_counts(dst_hbm, et_hbm, out_hbm, dst_sl, et_sl, hist):
    c = lax.axis_index("c")
    s = lax.axis_index("s")
    t = s * NC + c
    pltpu.sync_copy(dst_hbm.at[t], dst_sl)
    pltpu.sync_copy(et_hbm.at[t], et_sl)
    zero = jnp.zeros((LANES,), jnp.float32)

    def zrow(i, carry):
        hist[pl.ds(i * LANES, LANES)] = zero
        return carry

    lax.fori_loop(0, KROWS, zrow, 0)
    ones = jnp.ones((LANES,), jnp.float32)

    def chunk(ci, carry):
        for m in range(CHUNK // LANES):
            dv = dst_sl[ci, pl.ds(m * LANES, LANES)]
            ev = et_sl[ci, pl.ds(m * LANES, LANES)]
            kv = dv * R + ev
            cnt, last = plsc.scan_count(kv)
            plsc.addupdate_scatter(hist, [kv], cnt.astype(jnp.float32),
                                   mask=last)
        return carry

    lax.fori_loop(0, NCH32, chunk, 0)
    pltpu.sync_copy(hist, out_hbm.at[t])


def _tc_inv_body(h_ref, o_ref):
    tot = jnp.sum(h_ref[...], axis=0)
    o_ref[...] = 1.0 / jnp.maximum(tot, 1.0)


_tc_inv = pl.pallas_call(
    _tc_inv_body,
    grid=(KROWS * 16 // (128 * 128),),
    in_specs=[pl.BlockSpec((NW, 128, 128), lambda i: (0, i, 0))],
    out_specs=pl.BlockSpec((128, 128), lambda i: (i, 0)),
    out_shape=jax.ShapeDtypeStruct((KROWS * 16 // 128, 128), jnp.float32),
)


@functools.partial(
    pl.kernel,
    out_type=jax.ShapeDtypeStruct((NW, NCH32, CHUNK), jnp.float32),
    mesh=_MESH,
    compiler_params=_SC_PARAMS,
    scratch_types=[
        pltpu.VMEM((KROWS * 16,), jnp.float32),
        pltpu.VMEM((NCH32, CHUNK), jnp.int32),
        pltpu.VMEM((NCH32, CHUNK), jnp.int32),
        pltpu.VMEM((NCH32, CHUNK), jnp.float32),
    ],
)
def _sc_w(inv_hbm, dst_hbm, et_hbm, w_hbm, inv_v, dst_sl, et_sl, w_sl):
    c = lax.axis_index("c")
    s = lax.axis_index("s")
    t = s * NC + c
    pltpu.sync_copy(inv_hbm, inv_v)
    pltpu.sync_copy(dst_hbm.at[t], dst_sl)
    pltpu.sync_copy(et_hbm.at[t], et_sl)

    def chunk(ci, carry):
        for m in range(CHUNK // LANES):
            dv = dst_sl[ci, pl.ds(m * LANES, LANES)]
            ev = et_sl[ci, pl.ds(m * LANES, LANES)]
            kv = dv * R + ev
            wv = plsc.load_gather(inv_v, [kv])
            w_sl[ci, pl.ds(m * LANES, LANES)] = wv
        return carry

    lax.fori_loop(0, NCH32, chunk, 0)
    pltpu.sync_copy(w_sl, w_hbm.at[t])


def _make_sc_conv(DH, NPH):
    @functools.partial(
        pl.kernel,
        out_type=jax.ShapeDtypeStruct((NC, NPH, NS, RPS, DH), jnp.float32),
        mesh=_MESH,
        compiler_params=_SC_PARAMS,
        scratch_types=[
            pltpu.VMEM((NCH16, CHUNK), jnp.int32),
            pltpu.VMEM((NCH16, CHUNK), jnp.int32),
            pltpu.VMEM((NCH16, CHUNK), jnp.int32),
            pltpu.VMEM((NCH16, CHUNK), jnp.float32),
            pltpu.VMEM((NCH16, CHUNK), jnp.int32),
            pltpu.VMEM((2, CHUNK, DH), jnp.float32),
            pltpu.VMEM((ZB, DH), jnp.float32),
            pltpu.VMEM_SHARED((NPAD, DH), jnp.float32),
            pltpu.SemaphoreType.DMA,
            pltpu.SemaphoreType.DMA,
        ],
    )
    def conv(h_hbm, src_hbm, et_hbm, dst_hbm, w_hbm, out_hbm,
             src_sl, et_sl, dst_sl, w_sl, gidx_sl, rows, zbuf, agg,
             gsem, ssem):
        c = lax.axis_index("c")
        s = lax.axis_index("s")
        pltpu.sync_copy(src_hbm.at[s], src_sl)
        pltpu.sync_copy(et_hbm.at[s], et_sl)
        pltpu.sync_copy(dst_hbm.at[s], dst_sl)
        pltpu.sync_copy(w_hbm.at[s], w_sl)

        zero = jnp.zeros((LANES,), jnp.float32)

        def zrow(i, carry):
            for j in range(DH // LANES):
                zbuf[i, pl.ds(j * LANES, LANES)] = zero
            return carry

        lax.fori_loop(0, ZB, zrow, 0)

        def gidxrow(ci, carry):
            for m in range(CHUNK // LANES):
                sv = src_sl[ci, pl.ds(m * LANES, LANES)]
                ev = et_sl[ci, pl.ds(m * LANES, LANES)]
                gidx_sl[ci, pl.ds(m * LANES, LANES)] = sv * R + ev
            return carry

        lax.fori_loop(0, NCH16, gidxrow, 0)

        for q in range(NPH):
            base = (c * NPH + q) * TBASE
            tbl = h_hbm.at[pl.ds(base, TBASE)]
            for k in range(RPS // ZB):
                pltpu.sync_copy(zbuf, agg.at[pl.ds(s * RPS + k * ZB, ZB)])
            plsc.subcore_barrier()

            pltpu.async_copy(tbl.at[gidx_sl.at[0]], rows.at[0], gsem)

            def chunk(ci, carry):
                b = ci & 1

                @pl.when(ci > 0)
                def _wait_prev_scatter():
                    pltpu.make_async_copy(
                        rows.at[1 - b], agg.at[dst_sl.at[ci]], ssem).wait()

                @pl.when(ci + 1 < NCH16)
                def _prefetch_next():
                    pltpu.async_copy(tbl.at[gidx_sl.at[ci + 1]],
                                     rows.at[1 - b], gsem)

                pltpu.make_async_copy(tbl.at[gidx_sl.at[ci]],
                                      rows.at[b], gsem).wait()

                def sgroup(m, inner):
                    wv16 = w_sl[ci, pl.ds(m * LANES, LANES)]
                    for i in range(LANES):
                        wv = wv16[i]
                        row = m * LANES + i
                        for j in range(DH // LANES):
                            rows[b, row, pl.ds(j * LANES, LANES)] = (
                                rows[b, row, pl.ds(j * LANES, LANES)] * wv)
                    return inner

                lax.fori_loop(0, CHUNK // LANES, sgroup, 0)
                pltpu.async_copy(rows.at[b], agg.at[dst_sl.at[ci]], ssem,
                                 add=True)
                return carry

            lax.fori_loop(0, NCH16, chunk, 0)
            pltpu.make_async_copy(
                rows.at[(NCH16 - 1) & 1],
                agg.at[dst_sl.at[NCH16 - 1]], ssem).wait()
            plsc.subcore_barrier()
            pltpu.sync_copy(agg.at[pl.ds(s * RPS, RPS)], out_hbm.at[c, q, s])

    return conv


_sc_conv1 = _make_sc_conv(HID // 4, 2)
_sc_conv2 = _make_sc_conv(OUT // 2, 1)


def _tc_prep_body(b1_ref, c1_ref, r1_ref, b2_ref, c2_ref, r2_ref,
                  w1_ref, w2_ref):
    for r in range(R):
        acc = jnp.zeros((IN, HID), jnp.float32)
        for b in range(NB):
            acc = acc + c1_ref[r, b] * b1_ref[b]
        w1_ref[:, r * HID:(r + 1) * HID] = acc
    w1_ref[:, R * HID:] = r1_ref[...]
    for r in range(R):
        acc = jnp.zeros((HID, OUT), jnp.float32)
        for b in range(NB):
            acc = acc + c2_ref[r, b] * b2_ref[b]
        w2_ref[:, r * OUT:(r + 1) * OUT] = acc
    w2_ref[:, R * OUT:] = r2_ref[...]


_tc_prep = pl.pallas_call(
    _tc_prep_body,
    in_specs=[
        pl.BlockSpec(memory_space=pltpu.VMEM),
        pl.BlockSpec(memory_space=pltpu.SMEM),
        pl.BlockSpec(memory_space=pltpu.VMEM),
        pl.BlockSpec(memory_space=pltpu.VMEM),
        pl.BlockSpec(memory_space=pltpu.SMEM),
        pl.BlockSpec(memory_space=pltpu.VMEM),
    ],
    out_specs=[
        pl.BlockSpec(memory_space=pltpu.VMEM),
        pl.BlockSpec(memory_space=pltpu.VMEM),
    ],
    out_shape=[
        jax.ShapeDtypeStruct((IN, R * HID + HID), jnp.float32),
        jax.ShapeDtypeStruct((HID, R * OUT + OUT), jnp.float32),
    ],
)


def _tc_s1_body(sm_ref, sg_ref, wm_ref, wg_ref, bm_ref, bg_ref,
                wall_ref, b1_ref, tab_ref, self_ref):
    p = pl.program_id(0)
    i = pl.program_id(1)
    xm = jnp.dot(sm_ref[...], wm_ref[...],
                 preferred_element_type=jnp.float32) + bm_ref[...]
    xg = jnp.dot(sg_ref[...], wg_ref[...],
                 preferred_element_type=jnp.float32) + bg_ref[...]
    x = jnp.where(i < N_M // _BR, xm, xg)
    h = jnp.dot(x, wall_ref[...], preferred_element_type=jnp.float32)
    qw = HID // 4
    hm = h[:, :R * HID].reshape(_BR, R, HID)
    qs = [hm[:, :, q * qw:(q + 1) * qw] for q in range(4)]
    sel = jnp.where(p == 0, qs[0],
                    jnp.where(p == 1, qs[1],
                              jnp.where(p == 2, qs[2], qs[3])))
    tab_ref[...] = sel.reshape(_BR * R, qw)
    self_ref[...] = h[:, R * HID:] + b1_ref[...]


_tc_s1 = pl.pallas_call(
    _tc_s1_body,
    grid=(4, N // _BR),
    in_specs=[
        pl.BlockSpec((_BR, INIT),
                     lambda p, i: (jnp.minimum(i, N_M // _BR - 1), 0)),
        pl.BlockSpec((_BR, INIT),
                     lambda p, i: (jnp.maximum(i - N_M // _BR, 0), 0)),
        pl.BlockSpec((INIT, IN), lambda p, i: (0, 0)),
        pl.BlockSpec((INIT, IN), lambda p, i: (0, 0)),
        pl.BlockSpec((1, IN), lambda p, i: (0, 0)),
        pl.BlockSpec((1, IN), lambda p, i: (0, 0)),
        pl.BlockSpec((IN, R * HID + HID), lambda p, i: (0, 0)),
        pl.BlockSpec((1, HID), lambda p, i: (0, 0)),
    ],
    out_specs=[
        pl.BlockSpec((_BR * R, HID // 4),
                     lambda p, i: (p * (N // _BR) + i, 0)),
        pl.BlockSpec((_BR, HID), lambda p, i: (i, 0)),
    ],
    out_shape=[
        jax.ShapeDtypeStruct((4 * TBASE, HID // 4), jnp.float32),
        jax.ShapeDtypeStruct((N, HID), jnp.float32),
    ],
)


def _tc_s2_body(a_ref, s1_ref, wall_ref, b2_ref, tab_ref, self_ref):
    p = pl.program_id(0)
    x = s1_ref[...] + jnp.concatenate(
        [a_ref[0], a_ref[1], a_ref[2], a_ref[3]], axis=1)
    h = jnp.dot(x, wall_ref[...], preferred_element_type=jnp.float32)
    ow = OUT // 2
    hm = h[:, :R * OUT].reshape(_BR, R, OUT)
    sel = jnp.where(p == 0, hm[:, :, :ow], hm[:, :, ow:])
    tab_ref[...] = sel.reshape(_BR * R, ow)
    self_ref[...] = h[:, R * OUT:] + b2_ref[...]


_tc_s2 = pl.pallas_call(
    _tc_s2_body,
    grid=(NC, N // _BR),
    in_specs=[
        pl.BlockSpec((4, _BR, HID // 4), lambda p, i: (0, i, 0)),
        pl.BlockSpec((_BR, HID), lambda p, i: (i, 0)),
        pl.BlockSpec((HID, R * OUT + OUT), lambda p, i: (0, 0)),
        pl.BlockSpec((1, OUT), lambda p, i: (0, 0)),
    ],
    out_specs=[
        pl.BlockSpec((_BR * R, OUT // 2),
                     lambda p, i: (p * (N // _BR) + i, 0)),
        pl.BlockSpec((_BR, OUT), lambda p, i: (i, 0)),
    ],
    out_shape=[
        jax.ShapeDtypeStruct((NC * TBASE, OUT // 2), jnp.float32),
        jax.ShapeDtypeStruct((N, OUT), jnp.float32),
    ],
)


def _tc_s3_body(a_ref, s2_ref, o_ref):
    o_ref[...] = s2_ref[...] + jnp.concatenate(
        [a_ref[0], a_ref[1]], axis=1)


_tc_s3 = pl.pallas_call(
    _tc_s3_body,
    grid=(N // _BR,),
    in_specs=[
        pl.BlockSpec((NC, _BR, OUT // 2), lambda i: (0, i, 0)),
        pl.BlockSpec((_BR, OUT), lambda i: (i, 0)),
    ],
    out_specs=pl.BlockSpec((_BR, OUT), lambda i: (i, 0)),
    out_shape=jax.ShapeDtypeStruct((N, OUT), jnp.float32),
)


def kernel(sim_m, sim_g, edge_index, edge_type, W_m, b_m, W_g, b_g,
           bases1, comp1, root1, bias1, bases2, comp2, root2, bias2):
    src32 = edge_index[0].reshape(NW, NCH32, CHUNK)
    dst32 = edge_index[1].reshape(NW, NCH32, CHUNK)
    et32 = edge_type.reshape(NW, NCH32, CHUNK)
    src16 = edge_index[0].reshape(NS, NCH16, CHUNK)
    dst16 = edge_index[1].reshape(NS, NCH16, CHUNK)
    et16 = edge_type.reshape(NS, NCH16, CHUNK)

    hists = _sc_counts(dst32, et32)
    inv = _tc_inv(hists.reshape(NW, KROWS * 16 // 128, 128))
    w32 = _sc_w(inv.reshape(KROWS * 16), dst32, et32)
    w16 = w32.reshape(NS, NCH16, CHUNK)

    wall1, wall2 = _tc_prep(bases1, comp1, root1, bases2, comp2, root2)
    tab1, self1 = _tc_s1(sim_m, sim_g, W_m, W_g,
                         b_m.reshape(1, IN), b_g.reshape(1, IN),
                         wall1, bias1.reshape(1, HID))
    agg1 = _sc_conv1(tab1, src16, et16, dst16, w16)
    tab2, self2 = _tc_s2(agg1.reshape(4, NPAD, HID // 4), self1, wall2,
                         bias2.reshape(1, OUT))
    agg2 = _sc_conv2(tab2, src16, et16, dst16, w16)
    out = _tc_s3(agg2.reshape(NC, NPAD, OUT // 2), self2)
    return out[:N_M], out[N_M:]


# speedup vs baseline: 1.1913x; 1.1913x over previous
import functools

import jax
import jax.numpy as jnp
from jax import lax
from jax.experimental import pallas as pl
from jax.experimental.pallas import tpu as pltpu
from jax.experimental.pallas import tpu_sc as plsc

N_M = 5000
N_G = 5000
N = N_M + N_G
E = 320000
R = 8
NB = 10
INIT = 256
IN = 128
HID = 128
OUT = 64

NC = 2
NS = 16
LANES = 16
NW = NC * NS
CHUNK = 80
EPT32 = E // NW
NCH32 = EPT32 // CHUNK
EPT16 = E // NS
NCH16 = EPT16 // CHUNK
KROWS = 5120
TBASE = N * R
NPAD = 10240
RPS = NPAD // NS
ZB = 128
_BR = 1000

_MESH = plsc.VectorSubcoreMesh(core_axis_name="c", subcore_axis_name="s")
_SC_PARAMS = pltpu.CompilerParams(needs_layout_passes=False,
                                  use_tc_tiling_on_sc=False)


@functools.partial(
    pl.kernel,
    out_type=jax.ShapeDtypeStruct((NW, KROWS * 16), jnp.float32),
    mesh=_MESH,
    compiler_params=_SC_PARAMS,
    scratch_types=[
        pltpu.VMEM((NCH32, CHUNK), jnp.int32),
        pltpu.VMEM((NCH32, CHUNK), jnp.int32),
        pltpu.VMEM((KROWS * 16,), jnp.float32),
    ],
)
def _sc_counts(dst_hbm, et_hbm, out_hbm, dst_sl, et_sl, hist):
    c = lax.axis_index("c")
    s = lax.axis_index("s")
    t = s * NC + c
    pltpu.sync_copy(dst_hbm.at[t], dst_sl)
    pltpu.sync_copy(et_hbm.at[t], et_sl)
    zero = jnp.zeros((LANES,), jnp.float32)

    def zrow(i, carry):
        hist[pl.ds(i * LANES, LANES)] = zero
        return carry

    lax.fori_loop(0, KROWS, zrow, 0)
    ones = jnp.ones((LANES,), jnp.float32)

    def chunk(ci, carry):
        for m in range(CHUNK // LANES):
            dv = dst_sl[ci, pl.ds(m * LANES, LANES)]
            ev = et_sl[ci, pl.ds(m * LANES, LANES)]
            kv = dv * R + ev
            cnt, last = plsc.scan_count(kv)
            plsc.addupdate_scatter(hist, [kv], cnt.astype(jnp.float32),
                                   mask=last)
        return carry

    lax.fori_loop(0, NCH32, chunk, 0)
    pltpu.sync_copy(hist, out_hbm.at[t])


def _tc_inv_body(h_ref, o_ref):
    tot = jnp.sum(h_ref[...], axis=0)
    o_ref[...] = 1.0 / jnp.maximum(tot, 1.0)


_tc_inv = pl.pallas_call(
    _tc_inv_body,
    grid=(KROWS * 16 // (128 * 128),),
    in_specs=[pl.BlockSpec((NW, 128, 128), lambda i: (0, i, 0))],
    out_specs=pl.BlockSpec((128, 128), lambda i: (i, 0)),
    out_shape=jax.ShapeDtypeStruct((KROWS * 16 // 128, 128), jnp.float32),
)


@functools.partial(
    pl.kernel,
    out_type=jax.ShapeDtypeStruct((NW, NCH32, CHUNK), jnp.float32),
    mesh=_MESH,
    compiler_params=_SC_PARAMS,
    scratch_types=[
        pltpu.VMEM((KROWS * 16,), jnp.float32),
        pltpu.VMEM((NCH32, CHUNK), jnp.int32),
        pltpu.VMEM((NCH32, CHUNK), jnp.int32),
        pltpu.VMEM((NCH32, CHUNK), jnp.float32),
    ],
)
def _sc_w(inv_hbm, dst_hbm, et_hbm, w_hbm, inv_v, dst_sl, et_sl, w_sl):
    c = lax.axis_index("c")
    s = lax.axis_index("s")
    t = s * NC + c
    pltpu.sync_copy(inv_hbm, inv_v)
    pltpu.sync_copy(dst_hbm.at[t], dst_sl)
    pltpu.sync_copy(et_hbm.at[t], et_sl)

    def chunk(ci, carry):
        for m in range(CHUNK // LANES):
            dv = dst_sl[ci, pl.ds(m * LANES, LANES)]
            ev = et_sl[ci, pl.ds(m * LANES, LANES)]
            kv = dv * R + ev
            wv = plsc.load_gather(inv_v, [kv])
            w_sl[ci, pl.ds(m * LANES, LANES)] = wv
        return carry

    lax.fori_loop(0, NCH32, chunk, 0)
    pltpu.sync_copy(w_sl, w_hbm.at[t])


def _make_sc_conv(DH, NPH):
    @functools.partial(
        pl.kernel,
        out_type=jax.ShapeDtypeStruct((NC, NPH, NS, RPS, DH), jnp.float32),
        mesh=_MESH,
        compiler_params=_SC_PARAMS,
        scratch_types=[
            pltpu.VMEM((NCH16, CHUNK), jnp.int32),
            pltpu.VMEM((NCH16, CHUNK), jnp.int32),
            pltpu.VMEM((NCH16, CHUNK), jnp.int32),
            pltpu.VMEM((NCH16, CHUNK), jnp.float32),
            pltpu.VMEM((NCH16, CHUNK), jnp.int32),
            pltpu.VMEM((2, CHUNK, DH), jnp.float32),
            pltpu.VMEM((ZB, DH), jnp.float32),
            pltpu.VMEM_SHARED((NPAD, DH), jnp.float32),
            pltpu.SemaphoreType.DMA,
            pltpu.SemaphoreType.DMA,
        ],
    )
    def conv(h_hbm, src_hbm, et_hbm, dst_hbm, w_hbm, out_hbm,
             src_sl, et_sl, dst_sl, w_sl, gidx_sl, rows, zbuf, agg,
             gsem, ssem):
        c = lax.axis_index("c")
        s = lax.axis_index("s")
        pltpu.sync_copy(src_hbm.at[s], src_sl)
        pltpu.sync_copy(et_hbm.at[s], et_sl)
        pltpu.sync_copy(dst_hbm.at[s], dst_sl)
        pltpu.sync_copy(w_hbm.at[s], w_sl)

        zero = jnp.zeros((LANES,), jnp.float32)

        def zrow(i, carry):
            for j in range(DH // LANES):
                zbuf[i, pl.ds(j * LANES, LANES)] = zero
            return carry

        lax.fori_loop(0, ZB, zrow, 0)

        def gidxrow(ci, carry):
            for m in range(CHUNK // LANES):
                sv = src_sl[ci, pl.ds(m * LANES, LANES)]
                ev = et_sl[ci, pl.ds(m * LANES, LANES)]
                gidx_sl[ci, pl.ds(m * LANES, LANES)] = sv * R + ev
            return carry

        lax.fori_loop(0, NCH16, gidxrow, 0)

        for q in range(NPH):
            base = (c * NPH + q) * TBASE
            tbl = h_hbm.at[pl.ds(base, TBASE)]
            for k in range(RPS // ZB):
                pltpu.sync_copy(zbuf, agg.at[pl.ds(s * RPS + k * ZB, ZB)])
            plsc.subcore_barrier()

            pltpu.async_copy(tbl.at[gidx_sl.at[0]], rows.at[0], gsem)

            def chunk(ci, carry):
                b = ci & 1

                @pl.when(ci > 0)
                def _wait_prev_scatter():
                    pltpu.make_async_copy(
                        rows.at[1 - b], agg.at[dst_sl.at[ci]], ssem).wait()

                @pl.when(ci + 1 < NCH16)
                def _prefetch_next():
                    pltpu.async_copy(tbl.at[gidx_sl.at[ci + 1]],
                                     rows.at[1 - b], gsem)

                pltpu.make_async_copy(tbl.at[gidx_sl.at[ci]],
                                      rows.at[b], gsem).wait()

                def sgroup(m, inner):
                    wv16 = w_sl[ci, pl.ds(m * LANES, LANES)]
                    for i in range(LANES):
                        wv = wv16[i]
                        row = m * LANES + i
                        for j in range(DH // LANES):
                            rows[b, row, pl.ds(j * LANES, LANES)] = (
                                rows[b, row, pl.ds(j * LANES, LANES)] * wv)
                    return inner

                lax.fori_loop(0, CHUNK // LANES, sgroup, 0)
                pltpu.async_copy(rows.at[b], agg.at[dst_sl.at[ci]], ssem,
                                 add=True)
                return carry

            lax.fori_loop(0, NCH16, chunk, 0)
            pltpu.make_async_copy(
                rows.at[(NCH16 - 1) & 1],
                agg.at[dst_sl.at[NCH16 - 1]], ssem).wait()
            plsc.subcore_barrier()
            pltpu.sync_copy(agg.at[pl.ds(s * RPS, RPS)], out_hbm.at[c, q, s])

    return conv


_sc_conv1 = _make_sc_conv(HID // 4, 2)
_sc_conv2 = _make_sc_conv(OUT // 2, 1)


def _tc_prep_body(b1_ref, c1_ref, r1_ref, b2_ref, c2_ref, r2_ref,
                  w1_ref, w2_ref):
    for r in range(R):
        acc = jnp.zeros((IN, HID), jnp.float32)
        for b in range(NB):
            acc = acc + c1_ref[r, b] * b1_ref[b]
        w1_ref[:, r * HID:(r + 1) * HID] = acc
    w1_ref[:, R * HID:] = r1_ref[...]
    for r in range(R):
        acc = jnp.zeros((HID, OUT), jnp.float32)
        for b in range(NB):
            acc = acc + c2_ref[r, b] * b2_ref[b]
        w2_ref[:, r * OUT:(r + 1) * OUT] = acc
    w2_ref[:, R * OUT:] = r2_ref[...]


_tc_prep = pl.pallas_call(
    _tc_prep_body,
    in_specs=[
        pl.BlockSpec(memory_space=pltpu.VMEM),
        pl.BlockSpec(memory_space=pltpu.SMEM),
        pl.BlockSpec(memory_space=pltpu.VMEM),
        pl.BlockSpec(memory_space=pltpu.VMEM),
        pl.BlockSpec(memory_space=pltpu.SMEM),
        pl.BlockSpec(memory_space=pltpu.VMEM),
    ],
    out_specs=[
        pl.BlockSpec(memory_space=pltpu.VMEM),
        pl.BlockSpec(memory_space=pltpu.VMEM),
    ],
    out_shape=[
        jax.ShapeDtypeStruct((IN, R * HID + HID), jnp.float32),
        jax.ShapeDtypeStruct((HID, R * OUT + OUT), jnp.float32),
    ],
)


def _tc_s1_body(sm_ref, sg_ref, wm_ref, wg_ref, bm_ref, bg_ref,
                wall_ref, b1_ref, tab_ref, self_ref):
    p = pl.program_id(0)
    i = pl.program_id(1)
    xm = jnp.dot(sm_ref[...], wm_ref[...],
                 preferred_element_type=jnp.float32) + bm_ref[...]
    xg = jnp.dot(sg_ref[...], wg_ref[...],
                 preferred_element_type=jnp.float32) + bg_ref[...]
    x = jnp.where(i < N_M // _BR, xm, xg)
    h = jnp.dot(x, wall_ref[...], preferred_element_type=jnp.float32)
    qw = HID // 4
    for r in range(R):
        qs = [h[:, r * HID + q * qw:r * HID + (q + 1) * qw]
              for q in range(4)]
        sel = jnp.where(p == 0, qs[0],
                        jnp.where(p == 1, qs[1],
                                  jnp.where(p == 2, qs[2], qs[3])))
        tab_ref[:, r * qw:(r + 1) * qw] = sel
    self_ref[...] = h[:, R * HID:] + b1_ref[...]


_tc_s1 = pl.pallas_call(
    _tc_s1_body,
    grid=(4, N // _BR),
    in_specs=[
        pl.BlockSpec((_BR, INIT),
                     lambda p, i: (jnp.minimum(i, N_M // _BR - 1), 0)),
        pl.BlockSpec((_BR, INIT),
                     lambda p, i: (jnp.maximum(i - N_M // _BR, 0), 0)),
        pl.BlockSpec((INIT, IN), lambda p, i: (0, 0)),
        pl.BlockSpec((INIT, IN), lambda p, i: (0, 0)),
        pl.BlockSpec((1, IN), lambda p, i: (0, 0)),
        pl.BlockSpec((1, IN), lambda p, i: (0, 0)),
        pl.BlockSpec((IN, R * HID + HID), lambda p, i: (0, 0)),
        pl.BlockSpec((1, HID), lambda p, i: (0, 0)),
    ],
    out_specs=[
        pl.BlockSpec((_BR, R * (HID // 4)),
                     lambda p, i: (p * (N // _BR) + i, 0)),
        pl.BlockSpec((_BR, HID), lambda p, i: (i, 0)),
    ],
    out_shape=[
        jax.ShapeDtypeStruct((4 * N, R * (HID // 4)), jnp.float32),
        jax.ShapeDtypeStruct((N, HID), jnp.float32),
    ],
)


def _tc_s2_body(a_ref, s1_ref, wall_ref, b2_ref, tab_ref, self_ref):
    p = pl.program_id(0)
    x = s1_ref[...] + jnp.concatenate(
        [a_ref[0], a_ref[1], a_ref[2], a_ref[3]], axis=1)
    h = jnp.dot(x, wall_ref[...], preferred_element_type=jnp.float32)
    ow = OUT // 2
    for r in range(R):
        lo = h[:, r * OUT:r * OUT + ow]
        hi = h[:, r * OUT + ow:(r + 1) * OUT]
        tab_ref[:, r * ow:(r + 1) * ow] = jnp.where(p == 0, lo, hi)
    self_ref[...] = h[:, R * OUT:] + b2_ref[...]


_tc_s2 = pl.pallas_call(
    _tc_s2_body,
    grid=(NC, N // _BR),
    in_specs=[
        pl.BlockSpec((4, _BR, HID // 4), lambda p, i: (0, i, 0)),
        pl.BlockSpec((_BR, HID), lambda p, i: (i, 0)),
        pl.BlockSpec((HID, R * OUT + OUT), lambda p, i: (0, 0)),
        pl.BlockSpec((1, OUT), lambda p, i: (0, 0)),
    ],
    out_specs=[
        pl.BlockSpec((_BR, R * (OUT // 2)),
                     lambda p, i: (p * (N // _BR) + i, 0)),
        pl.BlockSpec((_BR, OUT), lambda p, i: (i, 0)),
    ],
    out_shape=[
        jax.ShapeDtypeStruct((NC * N, R * (OUT // 2)), jnp.float32),
        jax.ShapeDtypeStruct((N, OUT), jnp.float32),
    ],
)


def _tc_s3_body(a_ref, s2_ref, o_ref):
    o_ref[...] = s2_ref[...] + jnp.concatenate(
        [a_ref[0], a_ref[1]], axis=1)


_tc_s3 = pl.pallas_call(
    _tc_s3_body,
    grid=(N // _BR,),
    in_specs=[
        pl.BlockSpec((NC, _BR, OUT // 2), lambda i: (0, i, 0)),
        pl.BlockSpec((_BR, OUT), lambda i: (i, 0)),
    ],
    out_specs=pl.BlockSpec((_BR, OUT), lambda i: (i, 0)),
    out_shape=jax.ShapeDtypeStruct((N, OUT), jnp.float32),
)


def kernel(sim_m, sim_g, edge_index, edge_type, W_m, b_m, W_g, b_g,
           bases1, comp1, root1, bias1, bases2, comp2, root2, bias2):
    src32 = edge_index[0].reshape(NW, NCH32, CHUNK)
    dst32 = edge_index[1].reshape(NW, NCH32, CHUNK)
    et32 = edge_type.reshape(NW, NCH32, CHUNK)
    src16 = edge_index[0].reshape(NS, NCH16, CHUNK)
    dst16 = edge_index[1].reshape(NS, NCH16, CHUNK)
    et16 = edge_type.reshape(NS, NCH16, CHUNK)

    hists = _sc_counts(dst32, et32)
    inv = _tc_inv(hists.reshape(NW, KROWS * 16 // 128, 128))
    w32 = _sc_w(inv.reshape(KROWS * 16), dst32, et32)
    w16 = w32.reshape(NS, NCH16, CHUNK)

    wall1, wall2 = _tc_prep(bases1, comp1, root1, bases2, comp2, root2)
    tab1, self1 = _tc_s1(sim_m, sim_g, W_m, W_g,
                         b_m.reshape(1, IN), b_g.reshape(1, IN),
                         wall1, bias1.reshape(1, HID))
    agg1 = _sc_conv1(tab1.reshape(4 * TBASE, HID // 4),
                     src16, et16, dst16, w16)
    tab2, self2 = _tc_s2(agg1.reshape(4, NPAD, HID // 4), self1, wall2,
                         bias2.reshape(1, OUT))
    agg2 = _sc_conv2(tab2.reshape(NC * TBASE, OUT // 2),
                     src16, et16, dst16, w16)
    out = _tc_s3(agg2.reshape(NC, NPAD, OUT // 2), self2)
    return out[:N_M], out[N_M:]
